# baseline (speedup 1.0000x reference)
import functools

import jax
import jax.numpy as jnp
import numpy as np
from jax import lax
from jax.experimental import pallas as pl
from jax.experimental.pallas import tpu as pltpu
from jax.experimental.pallas import tpu_sc as plsc

N_CHK = 5000
N_VAR = 5000
E = 80000
H = 256
L = 10

NP = 5120
BLK = 512
GRID = NP // BLK
TILES = 16
RPT = NP // TILES
RPTP = 328
CHUNK = 64
CAP = 80128
NCHT = CAP // CHUNK
LANE = 16


def _init_body(syn_ref, wci_ref, bci_ref, bvi_ref, wvc_ref, bvc_ref,
               wcv_ref, bcv_ref, hc_ref, hv_ref, t_ref):
    syn = syn_ref[...]
    hc0 = syn * wci_ref[...] + bci_ref[...]
    hv0 = jnp.broadcast_to(bvi_ref[...], (BLK, H))
    hc_ref[...] = hc0
    hv_ref[...] = hv0
    t_ref[0] = jnp.tanh(jnp.dot(hv0, wvc_ref[...],
                                preferred_element_type=jnp.float32) + bvc_ref[...])
    t_ref[1] = jnp.tanh(jnp.dot(hc0, wcv_ref[...],
                                preferred_element_type=jnp.float32) + bcv_ref[...])


def _step_body(hc_ref, hv_ref, pc_ref, pv_ref, wnc_ref, bnc_ref, wnv_ref,
               bnv_ref, wvc_ref, bvc_ref, wcv_ref, bcv_ref,
               hcn_ref, hvn_ref, t_ref):
    hc = hc_ref[...]
    hv = hv_ref[...]
    cat_c = jnp.concatenate([hc, pc_ref[0]], axis=1)
    hcn = jnp.tanh(jnp.dot(cat_c, wnc_ref[...],
                           preferred_element_type=jnp.float32) + bnc_ref[...])
    cat_v = jnp.concatenate([hv, pv_ref[0]], axis=1)
    hvn = jnp.maximum(jnp.dot(cat_v, wnv_ref[...],
                              preferred_element_type=jnp.float32) + bnv_ref[...], 0.0)
    hcn_ref[...] = hcn
    hvn_ref[...] = hvn
    t_ref[0] = jnp.tanh(jnp.dot(hvn, wvc_ref[...],
                                preferred_element_type=jnp.float32) + bvc_ref[...])
    t_ref[1] = jnp.tanh(jnp.dot(hcn, wcv_ref[...],
                                preferred_element_type=jnp.float32) + bcv_ref[...])


def _final_body(hv_ref, wf_ref, bf_ref, out_ref):
    out_ref[...] = jax.nn.sigmoid(
        jnp.dot(hv_ref[...], wf_ref[...],
                preferred_element_type=jnp.float32) + bf_ref[...])


def _row_spec(shape):
    return pl.BlockSpec(shape, lambda i: (i,) + (0,) * (len(shape) - 1))


def _full_spec(shape):
    return pl.BlockSpec(shape, lambda i: (0,) * len(shape))


_init_call = pl.pallas_call(
    _init_body,
    grid=(GRID,),
    in_specs=[
        _row_spec((BLK, 1)),
        _full_spec((1, H)),
        _full_spec((1, H)),
        _full_spec((1, H)),
        _full_spec((H, H)),
        _full_spec((1, H)),
        _full_spec((H, H)),
        _full_spec((1, H)),
    ],
    out_specs=[
        _row_spec((BLK, H)),
        _row_spec((BLK, H)),
        pl.BlockSpec((2, BLK, H), lambda i: (0, i, 0)),
    ],
    out_shape=[
        jax.ShapeDtypeStruct((NP, H), jnp.float32),
        jax.ShapeDtypeStruct((NP, H), jnp.float32),
        jax.ShapeDtypeStruct((2, NP, H), jnp.float32),
    ],
)

_step_call = pl.pallas_call(
    _step_body,
    grid=(GRID,),
    in_specs=[
        _row_spec((BLK, H)),
        _row_spec((BLK, H)),
        pl.BlockSpec((1, BLK, H), lambda i: (0, i, 0)),
        pl.BlockSpec((1, BLK, H), lambda i: (1, i, 0)),
        _full_spec((2 * H, H)),
        _full_spec((1, H)),
        _full_spec((2 * H, H)),
        _full_spec((1, H)),
        _full_spec((H, H)),
        _full_spec((1, H)),
        _full_spec((H, H)),
        _full_spec((1, H)),
    ],
    out_specs=[
        _row_spec((BLK, H)),
        _row_spec((BLK, H)),
        pl.BlockSpec((2, BLK, H), lambda i: (0, i, 0)),
    ],
    out_shape=[
        jax.ShapeDtypeStruct((NP, H), jnp.float32),
        jax.ShapeDtypeStruct((NP, H), jnp.float32),
        jax.ShapeDtypeStruct((2, NP, H), jnp.float32),
    ],
)

_final_call = pl.pallas_call(
    _final_body,
    grid=(GRID,),
    in_specs=[
        _row_spec((BLK, H)),
        _full_spec((H, 1)),
        _full_spec((1, 1)),
    ],
    out_specs=_row_spec((BLK, 1)),
    out_shape=jax.ShapeDtypeStruct((NP, 1), jnp.float32),
)


def _sc_segsum_body(t_hbm, edges_hbm, counts_hbm, out_hbm,
                    src0, tgt0, src1, tgt1, cnt_v, g0, g1, acc_v, sem0, sem1):
    d = lax.axis_index("c")
    r = lax.axis_index("s")
    lanes = lax.iota(jnp.int32, LANE)
    zrow = jnp.zeros((LANE,), jnp.float32)
    bufs = ((src0, tgt0, g0, sem0), (src1, tgt1, g1, sem1))

    def zero_row(i, carry):
        rows = jnp.full((LANE,), i, jnp.int32)
        for k in range(H // LANE):
            plsc.store_scatter(acc_v, [rows, lanes + (k * LANE)], zrow)
        return carry

    lax.fori_loop(0, RPTP, zero_row, 0)

    pltpu.sync_copy(counts_hbm.at[d], cnt_v)
    cvec = plsc.load_gather(cnt_v, [jnp.full((LANE,), r, jnp.int32)])
    nch = lax.div(cvec[0] + (CHUNK - 1), CHUNK)

    def fetch(c, sv, tv_, gv, mv):
        pltpu.sync_copy(edges_hbm.at[d, 0, r, pl.ds(c * CHUNK, CHUNK)], sv)
        pltpu.sync_copy(edges_hbm.at[d, 1, r, pl.ds(c * CHUNK, CHUNK)], tv_)
        pltpu.async_copy(t_hbm.at[sv], gv, mv)

    def accumulate(tgt_v, g_v):
        for e in range(CHUNK):
            rows = plsc.load_gather(tgt_v, [jnp.full((LANE,), e, jnp.int32)])
            for k in range(H // LANE):
                plsc.addupdate_scatter(
                    acc_v, [rows, lanes + (k * LANE)],
                    g_v[e, pl.ds(k * LANE, LANE)])

    fetch(0, src0, tgt0, g0, sem0)

    def pair(jj, carry):
        for b in range(2):
            sb, tb, gb, mb = bufs[b]
            sn, tn, gn, mn = bufs[1 - b]
            c = jj * 2 + b
            fetch(lax.min(c + 1, NCHT - 1), sn, tn, gn, mn)
            pltpu.make_async_copy(t_hbm.at[sb], gb, mb).wait()
            accumulate(tb, gb)
        return carry

    lax.fori_loop(0, lax.div(nch + 1, 2), pair, 0)
    pltpu.make_async_copy(t_hbm.at[src0], g0, sem0).wait()
    pltpu.sync_copy(acc_v.at[pl.ds(0, RPT)],
                    out_hbm.at[d, pl.ds(r * RPT, RPT)])


@functools.lru_cache(maxsize=1)
def _sc_segsum():
    return pl.kernel(
        _sc_segsum_body,
        mesh=plsc.VectorSubcoreMesh(core_axis_name="c", subcore_axis_name="s"),
        compiler_params=pltpu.CompilerParams(needs_layout_passes=False),
        out_type=jax.ShapeDtypeStruct((2, NP, H), jnp.float32),
        scratch_types=[
            pltpu.VMEM((CHUNK,), jnp.int32),
            pltpu.VMEM((CHUNK,), jnp.int32),
            pltpu.VMEM((CHUNK,), jnp.int32),
            pltpu.VMEM((CHUNK,), jnp.int32),
            pltpu.VMEM((TILES,), jnp.int32),
            pltpu.VMEM((CHUNK, H), jnp.float32),
            pltpu.VMEM((CHUNK, H), jnp.float32),
            pltpu.VMEM((RPTP, H), jnp.float32),
            pltpu.SemaphoreType.DMA,
            pltpu.SemaphoreType.DMA,
        ],
    )


def _bucket_edges(src, tgt, src_off):
    b = tgt // RPT
    onehot = (b[None, :] == jnp.arange(TILES, dtype=jnp.int32)[:, None])
    counts = jnp.sum(onehot.astype(jnp.int32), axis=1)
    pos_all = jnp.cumsum(onehot.astype(jnp.int32), axis=1)
    pos = jnp.take_along_axis(pos_all, b[None, :], axis=0)[0] - 1
    flat = b * CAP + pos
    src_p = jnp.full((TILES * CAP,), src_off, jnp.int32).at[flat].set(
        src + src_off)
    tgt_p = jnp.full((TILES * CAP,), RPT, jnp.int32).at[flat].set(tgt - b * RPT)
    return (jnp.stack([src_p.reshape(TILES, CAP), tgt_p.reshape(TILES, CAP)]),
            counts)


def kernel(syndrome_s, c_to_v_sources, c_to_v_targets, v_to_c_sources,
           v_to_c_targets, W_check_in, b_check_in, W_var_in, b_var_in,
           W_msg_vc, b_msg_vc, W_next_check, b_next_check,
           W_msg_cv, b_msg_cv, W_next_var, b_next_var, W_final, b_final):
    syn = jnp.pad(syndrome_s, (0, NP - N_CHK)).reshape(NP, 1)
    bci = b_check_in.reshape(1, H)
    bvi = b_var_in.reshape(1, H)
    bvc = b_msg_vc.reshape(1, H)
    bcv = b_msg_cv.reshape(1, H)
    bnc = b_next_check.reshape(1, H)
    bnv = b_next_var.reshape(1, H)
    bf = b_final.reshape(1, 1)

    e0, c0 = _bucket_edges(v_to_c_sources, v_to_c_targets, 0)
    e1, c1 = _bucket_edges(c_to_v_sources, c_to_v_targets, NP)
    edges = jnp.stack([e0, e1])
    counts = jnp.stack([c0, c1])

    h_c, h_v, t_all = _init_call(syn, W_check_in, bci, bvi,
                                 W_msg_vc, bvc, W_msg_cv, bcv)
    for _ in range(L):
        pooled = _sc_segsum()(t_all.reshape(2 * NP, H), edges, counts)
        h_c, h_v, t_all = _step_call(h_c, h_v, pooled, pooled,
                                     W_next_check, bnc, W_next_var, bnv,
                                     W_msg_vc, bvc, W_msg_cv, bcv)
    pred = _final_call(h_v, W_final, bf)
    return pred[:N_VAR, 0]


# speedup vs baseline: 1.0567x; 1.0567x over previous
import functools

import jax
import jax.numpy as jnp
import numpy as np
from jax import lax
from jax.experimental import pallas as pl
from jax.experimental.pallas import tpu as pltpu
from jax.experimental.pallas import tpu_sc as plsc

N_CHK = 5000
N_VAR = 5000
E = 80000
H = 256
L = 10

NP = 5120
BLK = 512
GRID = NP // BLK
TILES = 16
RPT = NP // TILES
RPTP = 328
CHUNK = 64
CAP = 80128
NCHT = CAP // CHUNK
LANE = 16


def _init_body(syn_ref, wci_ref, bci_ref, bvi_ref, wvc_ref, bvc_ref,
               wcv_ref, bcv_ref, hc_ref, hv_ref, t_ref):
    syn = syn_ref[...]
    hc0 = syn * wci_ref[...] + bci_ref[...]
    hv0 = jnp.broadcast_to(bvi_ref[...], (BLK, H))
    hc_ref[...] = hc0
    hv_ref[...] = hv0
    t_ref[0] = jnp.tanh(jnp.dot(hv0, wvc_ref[...],
                                preferred_element_type=jnp.float32) + bvc_ref[...])
    t_ref[1] = jnp.tanh(jnp.dot(hc0, wcv_ref[...],
                                preferred_element_type=jnp.float32) + bcv_ref[...])


def _step_body(hc_ref, hv_ref, pc_ref, pv_ref, wnc_ref, bnc_ref, wnv_ref,
               bnv_ref, wvc_ref, bvc_ref, wcv_ref, bcv_ref,
               hcn_ref, hvn_ref, t_ref):
    hc = hc_ref[...]
    hv = hv_ref[...]
    cat_c = jnp.concatenate([hc, pc_ref[0]], axis=1)
    hcn = jnp.tanh(jnp.dot(cat_c, wnc_ref[...],
                           preferred_element_type=jnp.float32) + bnc_ref[...])
    cat_v = jnp.concatenate([hv, pv_ref[0]], axis=1)
    hvn = jnp.maximum(jnp.dot(cat_v, wnv_ref[...],
                              preferred_element_type=jnp.float32) + bnv_ref[...], 0.0)
    hcn_ref[...] = hcn
    hvn_ref[...] = hvn
    t_ref[0] = jnp.tanh(jnp.dot(hvn, wvc_ref[...],
                                preferred_element_type=jnp.float32) + bvc_ref[...])
    t_ref[1] = jnp.tanh(jnp.dot(hcn, wcv_ref[...],
                                preferred_element_type=jnp.float32) + bcv_ref[...])


def _final_body(hv_ref, wf_ref, bf_ref, out_ref):
    out_ref[...] = jax.nn.sigmoid(
        jnp.dot(hv_ref[...], wf_ref[...],
                preferred_element_type=jnp.float32) + bf_ref[...])


def _row_spec(shape):
    return pl.BlockSpec(shape, lambda i: (i,) + (0,) * (len(shape) - 1))


def _full_spec(shape):
    return pl.BlockSpec(shape, lambda i: (0,) * len(shape))


_init_call = pl.pallas_call(
    _init_body,
    grid=(GRID,),
    in_specs=[
        _row_spec((BLK, 1)),
        _full_spec((1, H)),
        _full_spec((1, H)),
        _full_spec((1, H)),
        _full_spec((H, H)),
        _full_spec((1, H)),
        _full_spec((H, H)),
        _full_spec((1, H)),
    ],
    out_specs=[
        _row_spec((BLK, H)),
        _row_spec((BLK, H)),
        pl.BlockSpec((2, BLK, H), lambda i: (0, i, 0)),
    ],
    out_shape=[
        jax.ShapeDtypeStruct((NP, H), jnp.float32),
        jax.ShapeDtypeStruct((NP, H), jnp.float32),
        jax.ShapeDtypeStruct((2, NP, H), jnp.float32),
    ],
)

_step_call = pl.pallas_call(
    _step_body,
    grid=(GRID,),
    in_specs=[
        _row_spec((BLK, H)),
        _row_spec((BLK, H)),
        pl.BlockSpec((1, BLK, H), lambda i: (0, i, 0)),
        pl.BlockSpec((1, BLK, H), lambda i: (1, i, 0)),
        _full_spec((2 * H, H)),
        _full_spec((1, H)),
        _full_spec((2 * H, H)),
        _full_spec((1, H)),
        _full_spec((H, H)),
        _full_spec((1, H)),
        _full_spec((H, H)),
        _full_spec((1, H)),
    ],
    out_specs=[
        _row_spec((BLK, H)),
        _row_spec((BLK, H)),
        pl.BlockSpec((2, BLK, H), lambda i: (0, i, 0)),
    ],
    out_shape=[
        jax.ShapeDtypeStruct((NP, H), jnp.float32),
        jax.ShapeDtypeStruct((NP, H), jnp.float32),
        jax.ShapeDtypeStruct((2, NP, H), jnp.float32),
    ],
)

_final_call = pl.pallas_call(
    _final_body,
    grid=(GRID,),
    in_specs=[
        _row_spec((BLK, H)),
        _full_spec((H, 1)),
        _full_spec((1, 1)),
    ],
    out_specs=_row_spec((BLK, 1)),
    out_shape=jax.ShapeDtypeStruct((NP, 1), jnp.float32),
)


def _sc_segsum_body(t_hbm, edges_hbm, counts_hbm, out_hbm,
                    idx0, idx1, cnt_v, g0, g1, acc_v, sem0, sem1):
    d = lax.axis_index("c")
    r = lax.axis_index("s")
    lanes = lax.iota(jnp.int32, LANE)
    zrow = jnp.zeros((LANE,), jnp.float32)
    bufs = ((idx0, g0, sem0), (idx1, g1, sem1))

    def zero_row(i, carry):
        rows = jnp.full((LANE,), i, jnp.int32)
        for k in range(H // LANE):
            plsc.store_scatter(acc_v, [rows, lanes + (k * LANE)], zrow)
        return carry

    lax.fori_loop(0, RPTP, zero_row, 0)

    pltpu.sync_copy(counts_hbm.at[d], cnt_v)
    cvec = plsc.load_gather(cnt_v, [jnp.full((LANE,), r, jnp.int32)])
    nch = lax.div(cvec[0] + (CHUNK - 1), CHUNK)

    def fetch(c, iv, gv, mv):
        pltpu.sync_copy(edges_hbm.at[d, r, c], iv)
        pltpu.async_copy(t_hbm.at[iv.at[0]], gv, mv)

    def accumulate(idx_v, g_v):
        for g in range(CHUNK // LANE):
            tv = idx_v[1, pl.ds(g * LANE, LANE)]
            for u in range(LANE):
                rows = jnp.full((LANE,), tv[u], jnp.int32)
                for k in range(H // LANE):
                    plsc.addupdate_scatter(
                        acc_v, [rows, lanes + (k * LANE)],
                        g_v[g * LANE + u, pl.ds(k * LANE, LANE)])

    fetch(0, idx0, g0, sem0)

    def pair(jj, carry):
        for b in range(2):
            ib, gb, mb = bufs[b]
            in_, gn, mn = bufs[1 - b]
            c = jj * 2 + b
            fetch(lax.min(c + 1, NCHT - 1), in_, gn, mn)
            pltpu.make_async_copy(t_hbm.at[ib.at[0]], gb, mb).wait()
            accumulate(ib, gb)
        return carry

    lax.fori_loop(0, lax.div(nch + 1, 2), pair, 0)
    pltpu.make_async_copy(t_hbm.at[idx0.at[0]], g0, sem0).wait()
    pltpu.sync_copy(acc_v.at[pl.ds(0, RPT)],
                    out_hbm.at[d, pl.ds(r * RPT, RPT)])


@functools.lru_cache(maxsize=1)
def _sc_segsum():
    return pl.kernel(
        _sc_segsum_body,
        mesh=plsc.VectorSubcoreMesh(core_axis_name="c", subcore_axis_name="s"),
        compiler_params=pltpu.CompilerParams(needs_layout_passes=False),
        out_type=jax.ShapeDtypeStruct((2, NP, H), jnp.float32),
        scratch_types=[
            pltpu.VMEM((2, CHUNK), jnp.int32),
            pltpu.VMEM((2, CHUNK), jnp.int32),
            pltpu.VMEM((TILES,), jnp.int32),
            pltpu.VMEM((CHUNK, H), jnp.float32),
            pltpu.VMEM((CHUNK, H), jnp.float32),
            pltpu.VMEM((RPTP, H), jnp.float32),
            pltpu.SemaphoreType.DMA,
            pltpu.SemaphoreType.DMA,
        ],
    )


def _bucket_edges(src, tgt, src_off):
    b = tgt // RPT
    onehot = (b[None, :] == jnp.arange(TILES, dtype=jnp.int32)[:, None])
    counts = jnp.sum(onehot.astype(jnp.int32), axis=1)
    pos_all = jnp.cumsum(onehot.astype(jnp.int32), axis=1)
    pos = jnp.take_along_axis(pos_all, b[None, :], axis=0)[0] - 1
    flat = b * CAP + pos
    src_p = jnp.full((TILES * CAP,), src_off, jnp.int32).at[flat].set(
        src + src_off)
    tgt_p = jnp.full((TILES * CAP,), RPT, jnp.int32).at[flat].set(tgt - b * RPT)
    return (jnp.stack([src_p.reshape(TILES, NCHT, CHUNK),
                       tgt_p.reshape(TILES, NCHT, CHUNK)], axis=2),
            counts)


def kernel(syndrome_s, c_to_v_sources, c_to_v_targets, v_to_c_sources,
           v_to_c_targets, W_check_in, b_check_in, W_var_in, b_var_in,
           W_msg_vc, b_msg_vc, W_next_check, b_next_check,
           W_msg_cv, b_msg_cv, W_next_var, b_next_var, W_final, b_final):
    syn = jnp.pad(syndrome_s, (0, NP - N_CHK)).reshape(NP, 1)
    bci = b_check_in.reshape(1, H)
    bvi = b_var_in.reshape(1, H)
    bvc = b_msg_vc.reshape(1, H)
    bcv = b_msg_cv.reshape(1, H)
    bnc = b_next_check.reshape(1, H)
    bnv = b_next_var.reshape(1, H)
    bf = b_final.reshape(1, 1)

    e0, c0 = _bucket_edges(v_to_c_sources, v_to_c_targets, 0)
    e1, c1 = _bucket_edges(c_to_v_sources, c_to_v_targets, NP)
    edges = jnp.stack([e0, e1])
    counts = jnp.stack([c0, c1])

    h_c, h_v, t_all = _init_call(syn, W_check_in, bci, bvi,
                                 W_msg_vc, bvc, W_msg_cv, bcv)
    for _ in range(L):
        pooled = _sc_segsum()(t_all.reshape(2 * NP, H), edges, counts)
        h_c, h_v, t_all = _step_call(h_c, h_v, pooled, pooled,
                                     W_next_check, bnc, W_next_var, bnv,
                                     W_msg_vc, bvc, W_msg_cv, bcv)
    pred = _final_call(h_v, W_final, bf)
    return pred[:N_VAR, 0]
